# baseline (speedup 1.0000x reference)
import jax
import jax.numpy as jnp
from jax import lax
from jax.experimental import pallas as pl
from jax.experimental.pallas import tpu as pltpu
from jax.experimental.pallas import tpu_sc as plsc

N = 10000
E = 320000
F = 128
H = 64

NC = 2
NS = 16
NW = NC * NS
K = 128
CHUNKS = 80
EPT = CHUNKS * K
EP = NW * EPT
ACC_N = 10112
RPT = ACC_N // NS

_MESH = plsc.VectorSubcoreMesh(core_axis_name="c", subcore_axis_name="s")
_SC_PARAMS = pltpu.CompilerParams(use_tc_tiling_on_sc=False)
_SC_PARAMS_NOLAYOUT = pltpu.CompilerParams(use_tc_tiling_on_sc=False,
                                           needs_layout_passes=False)


def _sc_hist(cols3, zero16):
    def body(cols_hbm, zero_hbm, out_hbm, colbuf, ones_v, hist):
        c = lax.axis_index("c")
        s = lax.axis_index("s")
        tl = c * NS + s
        pltpu.sync_copy(cols_hbm.at[tl], colbuf)
        pltpu.sync_copy(zero_hbm.at[pl.ds(s * RPT, RPT)],
                        hist.at[pl.ds(s * RPT, RPT)])

        @pl.loop(0, K)
        def _(r):
            ones_v[r, :] = jnp.full((16,), 1.0, jnp.float32)

        plsc.subcore_barrier()

        @pl.loop(0, CHUNKS)
        def _(j):
            pltpu.sync_copy(ones_v, hist.at[colbuf.at[j]], add=True)

        plsc.subcore_barrier()
        pltpu.sync_copy(hist.at[pl.ds(s * RPT, RPT)],
                        out_hbm.at[c, pl.ds(s * RPT, RPT)])

    return pl.kernel(
        body,
        out_type=jax.ShapeDtypeStruct((NC, ACC_N, 16), jnp.float32),
        mesh=_MESH,
        compiler_params=_SC_PARAMS,
        scratch_types=[
            pltpu.VMEM((CHUNKS, K), jnp.int32),
            pltpu.VMEM((K, 16), jnp.float32),
            pltpu.VMEM_SHARED((ACC_N, 16), jnp.float32),
        ],
    )(cols3, zero16)


def _sc_agg(g, pos_ei, cols3, zero64):
    NBUF = 4

    EREAL = E // NW
    EFULL = (EREAL // K) * K

    def body(g_hbm, pos_hbm, cols_hbm, zero_hbm, out_hbm,
             rowbuf, colbuf, vb0, vb1, vb2, vb3, acc, sm0, sm1, sm2, sm3):
        c = lax.axis_index("c")
        s = lax.axis_index("s")
        tl = c * NS + s
        vbufs = (vb0, vb1, vb2, vb3)
        sems = (sm0, sm1, sm2, sm3)
        lanes = lax.iota(jnp.int32, 16)
        base = tl * EREAL
        pltpu.sync_copy(pos_hbm.at[0, pl.ds(base, EFULL)],
                        rowbuf.at[pl.ds(0, EFULL)])
        pltpu.sync_copy(pos_hbm.at[0, pl.ds(base + EFULL, EREAL - EFULL)],
                        rowbuf.at[pl.ds(EFULL, EREAL - EFULL)])
        for k in range((EPT - EREAL) // 16):
            rowbuf[pl.ds(EREAL + k * 16, 16)] = k * 16 + lanes
        pltpu.sync_copy(cols_hbm.at[tl], colbuf)
        pltpu.sync_copy(zero_hbm.at[pl.ds(s * RPT, RPT)],
                        acc.at[pl.ds(s * RPT, RPT)])
        plsc.subcore_barrier()

        for b in range(NBUF):
            pltpu.async_copy(g_hbm.at[rowbuf.at[pl.ds(b * K, K)]],
                             vbufs[b], sems[b])

        @pl.loop(0, CHUNKS // NBUF)
        def _(t):
            j = NBUF * t
            for b in range(NBUF):
                pltpu.make_async_copy(
                    g_hbm.at[rowbuf.at[pl.ds((j + b) * K, K)]],
                    vbufs[b], sems[b]).wait()
                pltpu.sync_copy(vbufs[b], acc.at[colbuf.at[j + b]], add=True)

                @pl.when(j + b + NBUF < CHUNKS)
                def _():
                    pltpu.async_copy(
                        g_hbm.at[rowbuf.at[pl.ds((j + b + NBUF) * K, K)]],
                        vbufs[b], sems[b])

        plsc.subcore_barrier()
        pltpu.sync_copy(acc.at[pl.ds(s * RPT, RPT)],
                        out_hbm.at[pl.ds(s * RPT, RPT), pl.ds(c * H, H)])

    return pl.kernel(
        body,
        out_type=jax.ShapeDtypeStruct((ACC_N, 2 * H), jnp.float32),
        mesh=_MESH,
        compiler_params=_SC_PARAMS,
        scratch_types=[
            pltpu.VMEM((EPT,), jnp.int32),
            pltpu.VMEM((CHUNKS, K), jnp.int32),
            pltpu.VMEM((K, H), jnp.float32),
            pltpu.VMEM((K, H), jnp.float32),
            pltpu.VMEM((K, H), jnp.float32),
            pltpu.VMEM((K, H), jnp.float32),
            pltpu.VMEM_SHARED((ACC_N, H), jnp.float32),
            pltpu.SemaphoreType.DMA,
            pltpu.SemaphoreType.DMA,
            pltpu.SemaphoreType.DMA,
            pltpu.SemaphoreType.DMA,
        ],
    )(g, pos_ei, cols3, zero64)


def _sc_dot(z, xo, ei):
    NBUF = 4

    EREAL = E // NW
    EFULL = (EREAL // K) * K

    def body(z_hbm, x_hbm, ei_hbm, out_hbm,
             i0buf, i1buf, zb0, zb1, zb2, zb3, xb0, xb1, xb2, xb3,
             ybuf, zs0, zs1, zs2, zs3, xs0, xs1, xs2, xs3):
        c = lax.axis_index("c")
        s = lax.axis_index("s")
        tl = c * NS + s
        zbufs = (zb0, zb1, zb2, zb3)
        xbufs = (xb0, xb1, xb2, xb3)
        zsems = (zs0, zs1, zs2, zs3)
        xsems = (xs0, xs1, xs2, xs3)
        lanes = lax.iota(jnp.int32, 16)
        base = tl * EREAL
        for r, buf in ((0, i0buf), (1, i1buf)):
            pltpu.sync_copy(ei_hbm.at[r, pl.ds(base, EFULL)],
                            buf.at[pl.ds(0, EFULL)])
            pltpu.sync_copy(ei_hbm.at[r, pl.ds(base + EFULL, EREAL - EFULL)],
                            buf.at[pl.ds(EFULL, EREAL - EFULL)])
            for k in range((EPT - EREAL) // 16):
                buf[pl.ds(EREAL + k * 16, 16)] = k * 16 + lanes
        for b in range(NBUF):
            pltpu.async_copy(z_hbm.at[i0buf.at[pl.ds(b * K, K)]],
                             zbufs[b], zsems[b])
            pltpu.async_copy(x_hbm.at[i1buf.at[pl.ds(b * K, K)]],
                             xbufs[b], xsems[b])

        @pl.loop(0, CHUNKS // NBUF)
        def _(t):
            j = NBUF * t
            for b in range(NBUF):
                zb, xb = zbufs[b], xbufs[b]
                pltpu.make_async_copy(z_hbm.at[i0buf.at[pl.ds((j + b) * K, K)]],
                                      zb, zsems[b]).wait()
                pltpu.make_async_copy(x_hbm.at[i1buf.at[pl.ds((j + b) * K, K)]],
                                      xb, xsems[b]).wait()

                @pl.loop(0, K // 16)
                def _(grp):
                    rows = grp * 16 + lanes
                    acc = jnp.zeros((16,), jnp.float32)
                    cols = lanes
                    for _f in range(H):
                        zv = plsc.load_gather(zb, [rows, cols])
                        xv = plsc.load_gather(xb, [rows, cols])
                        acc = acc + zv * xv
                        cols = (cols + 1) & (H - 1)
                    ybuf[j + b, pl.ds(grp * 16, 16)] = acc

                @pl.when(j + b + NBUF < CHUNKS)
                def _():
                    pltpu.async_copy(
                        z_hbm.at[i0buf.at[pl.ds((j + b + NBUF) * K, K)]],
                        zb, zsems[b])
                    pltpu.async_copy(
                        x_hbm.at[i1buf.at[pl.ds((j + b + NBUF) * K, K)]],
                        xb, xsems[b])

        pltpu.sync_copy(ybuf, out_hbm.at[tl])

    return pl.kernel(
        body,
        out_type=jax.ShapeDtypeStruct((NW, CHUNKS, K), jnp.float32),
        mesh=_MESH,
        compiler_params=_SC_PARAMS_NOLAYOUT,
        scratch_types=(
            [pltpu.VMEM((EPT,), jnp.int32)] * 2
            + [pltpu.VMEM((K, H), jnp.float32)] * 8
            + [pltpu.VMEM((CHUNKS, K), jnp.float32)]
            + [pltpu.SemaphoreType.DMA] * 8
        ),
    )(z, xo, ei)


def _dot(a, b):
    return lax.dot_general(a, b, (((1,), (0,)), ((), ())),
                           precision=lax.Precision.HIGHEST,
                           preferred_element_type=jnp.float32)


def _tc_mm1(x, W_init, b_init2, W1):
    def body(x_ref, wi_ref, bi_ref, w1_ref, o_ref):
        h = _dot(x_ref[...], wi_ref[...]) + bi_ref[...]
        o_ref[...] = _dot(h, w1_ref[...])

    return pl.pallas_call(
        body, out_shape=jax.ShapeDtypeStruct((N, H), jnp.float32),
    )(x, W_init, b_init2, W1)


def _tc_scale(h1, hist):
    def body(h_ref, hist_ref, g_ref, dinv_ref):
        deg = hist_ref[0, :N, 0:1] + hist_ref[1, :N, 0:1] + 1.0
        dinv = lax.rsqrt(deg)
        dinv_ref[...] = dinv
        g_ref[...] = h_ref[...] * dinv

    return pl.pallas_call(
        body,
        out_shape=(jax.ShapeDtypeStruct((N, H), jnp.float32),
                   jax.ShapeDtypeStruct((N, 1), jnp.float32)),
    )(h1, hist)


def _tc_mid(s_il, g1, dinv, b1_2, W2):
    def body(s_ref, g_ref, d_ref, b_ref, w_ref, o_ref):
        d = d_ref[...]
        ssum = s_ref[:N, :H] + s_ref[:N, H:]
        t = d * (ssum + g_ref[...]) + b_ref[...]
        r = jnp.maximum(t, 0.0)
        o_ref[...] = _dot(r, w_ref[...]) * d

    return pl.pallas_call(
        body, out_shape=jax.ShapeDtypeStruct((N, H), jnp.float32),
    )(s_il, g1, dinv, b1_2, W2)


def _tc_fin(s_il, g2, dinv, b2_2, Wb):
    def body(s_ref, g_ref, d_ref, b_ref, w_ref, t_ref, z_ref):
        ssum = s_ref[:N, :H] + s_ref[:N, H:]
        t = d_ref[...] * (ssum + g_ref[...]) + b_ref[...]
        t_ref[...] = t
        z_ref[...] = _dot(t, w_ref[...])

    return pl.pallas_call(
        body,
        out_shape=(jax.ShapeDtypeStruct((N, H), jnp.float32),
                   jax.ShapeDtypeStruct((N, H), jnp.float32)),
    )(s_il, g2, dinv, b2_2, Wb)


def _block_cols(idx, pad_dst):
    real = idx.reshape(NW, E // NW)
    padb = jnp.broadcast_to(pad_dst, (NW, EPT - E // NW))
    return jnp.concatenate([real, padb], axis=1).reshape(NW, CHUNKS, K)


def kernel(x_input, edge_index_input, pos_edge_index_input,
           W_init, b_init, W1, b1, W2, b2, W_bil, b_bil):
    pad_iota = jnp.arange(EPT - E // NW, dtype=jnp.int32)
    pad_dst = N + pad_iota % (ACC_N - N)
    cols3 = _block_cols(pos_edge_index_input[1], pad_dst)
    zero16 = jnp.zeros((ACC_N, 16), jnp.float32)
    zero64 = jnp.zeros((ACC_N, H), jnp.float32)

    hist = _sc_hist(cols3, zero16)
    h1 = _tc_mm1(x_input, W_init, b_init.reshape(1, F), W1)
    g1, dinv = _tc_scale(h1, hist)

    s1 = _sc_agg(g1, pos_edge_index_input, cols3, zero64)
    g2 = _tc_mid(s1, g1, dinv, b1.reshape(1, H), W2)

    s2 = _sc_agg(g2, pos_edge_index_input, cols3, zero64)
    out2, z = _tc_fin(s2, g2, dinv, b2.reshape(1, H), W_bil[0])

    yp = _sc_dot(z, out2, edge_index_input)
    return yp.reshape(NW, EPT)[:, :E // NW].reshape(-1) + b_bil[0]


# speedup vs baseline: 1.0191x; 1.0191x over previous
import jax
import jax.numpy as jnp
from jax import lax
from jax.experimental import pallas as pl
from jax.experimental.pallas import tpu as pltpu
from jax.experimental.pallas import tpu_sc as plsc

N = 10000
E = 320000
F = 128
H = 64

NC = 2
NS = 16
NW = NC * NS
K = 128
CHUNKS = 80
EPT = CHUNKS * K
EP = NW * EPT
ACC_N = 10112
RPT = ACC_N // NS

_MESH = plsc.VectorSubcoreMesh(core_axis_name="c", subcore_axis_name="s")
_SC_PARAMS = pltpu.CompilerParams(use_tc_tiling_on_sc=False)
_SC_PARAMS_NOLAYOUT = pltpu.CompilerParams(use_tc_tiling_on_sc=False,
                                           needs_layout_passes=False)


def _sc_hist(cols3, zero16):
    def body(cols_hbm, zero_hbm, out_hbm, colbuf, ones_v, hist):
        c = lax.axis_index("c")
        s = lax.axis_index("s")
        tl = c * NS + s
        pltpu.sync_copy(cols_hbm.at[tl], colbuf)
        pltpu.sync_copy(zero_hbm.at[pl.ds(s * RPT, RPT)],
                        hist.at[pl.ds(s * RPT, RPT)])

        @pl.loop(0, K)
        def _(r):
            ones_v[r, :] = jnp.full((16,), 1.0, jnp.float32)

        plsc.subcore_barrier()

        @pl.loop(0, CHUNKS)
        def _(j):
            pltpu.sync_copy(ones_v, hist.at[colbuf.at[j]], add=True)

        plsc.subcore_barrier()
        pltpu.sync_copy(hist.at[pl.ds(s * RPT, RPT)],
                        out_hbm.at[c, pl.ds(s * RPT, RPT)])

    return pl.kernel(
        body,
        out_type=jax.ShapeDtypeStruct((NC, ACC_N, 16), jnp.float32),
        mesh=_MESH,
        compiler_params=_SC_PARAMS,
        scratch_types=[
            pltpu.VMEM((CHUNKS, K), jnp.int32),
            pltpu.VMEM((K, 16), jnp.float32),
            pltpu.VMEM_SHARED((ACC_N, 16), jnp.float32),
        ],
    )(cols3, zero16)


def _sc_agg(g, rows3, cols3, zero64):
    NBUF = 4

    def body(g_hbm, rows_hbm, cols_hbm, zero_hbm, out_hbm,
             rowbuf, colbuf, vb0, vb1, vb2, vb3, acc, sm0, sm1, sm2, sm3):
        c = lax.axis_index("c")
        s = lax.axis_index("s")
        tl = c * NS + s
        vbufs = (vb0, vb1, vb2, vb3)
        sems = (sm0, sm1, sm2, sm3)
        pltpu.sync_copy(rows_hbm.at[tl], rowbuf)
        pltpu.sync_copy(cols_hbm.at[tl], colbuf)
        pltpu.sync_copy(zero_hbm.at[pl.ds(s * RPT, RPT)],
                        acc.at[pl.ds(s * RPT, RPT)])
        plsc.subcore_barrier()

        for b in range(NBUF):
            pltpu.async_copy(g_hbm.at[rowbuf.at[b]], vbufs[b], sems[b])

        @pl.loop(0, CHUNKS // NBUF)
        def _(t):
            j = NBUF * t
            for b in range(NBUF):
                pltpu.make_async_copy(g_hbm.at[rowbuf.at[j + b]],
                                      vbufs[b], sems[b]).wait()
                pltpu.sync_copy(vbufs[b], acc.at[colbuf.at[j + b]], add=True)

                @pl.when(j + b + NBUF < CHUNKS)
                def _():
                    pltpu.async_copy(g_hbm.at[rowbuf.at[j + b + NBUF]],
                                     vbufs[b], sems[b])

        plsc.subcore_barrier()
        pltpu.sync_copy(acc.at[pl.ds(s * RPT, RPT)],
                        out_hbm.at[pl.ds(s * RPT, RPT), pl.ds(c * H, H)])

    return pl.kernel(
        body,
        out_type=jax.ShapeDtypeStruct((ACC_N, 2 * H), jnp.float32),
        mesh=_MESH,
        compiler_params=_SC_PARAMS,
        scratch_types=[
            pltpu.VMEM((CHUNKS, K), jnp.int32),
            pltpu.VMEM((CHUNKS, K), jnp.int32),
            pltpu.VMEM((K, H), jnp.float32),
            pltpu.VMEM((K, H), jnp.float32),
            pltpu.VMEM((K, H), jnp.float32),
            pltpu.VMEM((K, H), jnp.float32),
            pltpu.VMEM_SHARED((ACC_N, H), jnp.float32),
            pltpu.SemaphoreType.DMA,
            pltpu.SemaphoreType.DMA,
            pltpu.SemaphoreType.DMA,
            pltpu.SemaphoreType.DMA,
        ],
    )(g, rows3, cols3, zero64)


def _sc_dot(z, xo, i0_3, i1_3):
    NBUF = 4

    def body(z_hbm, x_hbm, i0_hbm, i1_hbm, out_hbm,
             i0buf, i1buf, zb0, zb1, zb2, zb3, xb0, xb1, xb2, xb3,
             ybuf, zs0, zs1, zs2, zs3, xs0, xs1, xs2, xs3):
        c = lax.axis_index("c")
        s = lax.axis_index("s")
        tl = c * NS + s
        zbufs = (zb0, zb1, zb2, zb3)
        xbufs = (xb0, xb1, xb2, xb3)
        zsems = (zs0, zs1, zs2, zs3)
        xsems = (xs0, xs1, xs2, xs3)
        lanes = lax.iota(jnp.int32, 16)
        pltpu.sync_copy(i0_hbm.at[tl], i0buf)
        pltpu.sync_copy(i1_hbm.at[tl], i1buf)
        for b in range(NBUF):
            pltpu.async_copy(z_hbm.at[i0buf.at[b]], zbufs[b], zsems[b])
            pltpu.async_copy(x_hbm.at[i1buf.at[b]], xbufs[b], xsems[b])

        @pl.loop(0, CHUNKS // NBUF)
        def _(t):
            j = NBUF * t
            for b in range(NBUF):
                zb, xb = zbufs[b], xbufs[b]
                pltpu.make_async_copy(z_hbm.at[i0buf.at[j + b]],
                                      zb, zsems[b]).wait()
                pltpu.make_async_copy(x_hbm.at[i1buf.at[j + b]],
                                      xb, xsems[b]).wait()

                @pl.loop(0, K // 16)
                def _(grp):
                    rows = grp * 16 + lanes
                    acc = jnp.zeros((16,), jnp.float32)
                    cols = lanes
                    for _f in range(H):
                        zv = plsc.load_gather(zb, [rows, cols])
                        xv = plsc.load_gather(xb, [rows, cols])
                        acc = acc + zv * xv
                        cols = (cols + 1) & (H - 1)
                    ybuf[j + b, pl.ds(grp * 16, 16)] = acc

                @pl.when(j + b + NBUF < CHUNKS)
                def _():
                    pltpu.async_copy(z_hbm.at[i0buf.at[j + b + NBUF]],
                                     zb, zsems[b])
                    pltpu.async_copy(x_hbm.at[i1buf.at[j + b + NBUF]],
                                     xb, xsems[b])

        pltpu.sync_copy(ybuf, out_hbm.at[tl])

    return pl.kernel(
        body,
        out_type=jax.ShapeDtypeStruct((NW, CHUNKS, K), jnp.float32),
        mesh=_MESH,
        compiler_params=_SC_PARAMS_NOLAYOUT,
        scratch_types=(
            [pltpu.VMEM((CHUNKS, K), jnp.int32)] * 2
            + [pltpu.VMEM((K, H), jnp.float32)] * 8
            + [pltpu.VMEM((CHUNKS, K), jnp.float32)]
            + [pltpu.SemaphoreType.DMA] * 8
        ),
    )(z, xo, i0_3, i1_3)


def _dot(a, b):
    return lax.dot_general(a, b, (((1,), (0,)), ((), ())),
                           precision=lax.Precision.HIGHEST,
                           preferred_element_type=jnp.float32)


def _tc_mm1(x, W_init, b_init2, W1):
    def body(x_ref, wi_ref, bi_ref, w1_ref, o_ref):
        h = _dot(x_ref[...], wi_ref[...]) + bi_ref[...]
        o_ref[...] = _dot(h, w1_ref[...])

    return pl.pallas_call(
        body, out_shape=jax.ShapeDtypeStruct((N, H), jnp.float32),
    )(x, W_init, b_init2, W1)


def _tc_scale(h1, hist):
    def body(h_ref, hist_ref, g_ref, dinv_ref):
        deg = hist_ref[0, :N, 0:1] + hist_ref[1, :N, 0:1] + 1.0
        dinv = lax.rsqrt(deg)
        dinv_ref[...] = dinv
        g_ref[...] = h_ref[...] * dinv

    return pl.pallas_call(
        body,
        out_shape=(jax.ShapeDtypeStruct((N, H), jnp.float32),
                   jax.ShapeDtypeStruct((N, 1), jnp.float32)),
    )(h1, hist)


def _tc_mid(s_il, g1, dinv, b1_2, W2):
    def body(s_ref, g_ref, d_ref, b_ref, w_ref, o_ref):
        d = d_ref[...]
        ssum = s_ref[:N, :H] + s_ref[:N, H:]
        t = d * (ssum + g_ref[...]) + b_ref[...]
        r = jnp.maximum(t, 0.0)
        o_ref[...] = _dot(r, w_ref[...]) * d

    return pl.pallas_call(
        body, out_shape=jax.ShapeDtypeStruct((N, H), jnp.float32),
    )(s_il, g1, dinv, b1_2, W2)


def _tc_fin(s_il, g2, dinv, b2_2, Wb):
    def body(s_ref, g_ref, d_ref, b_ref, w_ref, t_ref, z_ref):
        ssum = s_ref[:N, :H] + s_ref[:N, H:]
        t = d_ref[...] * (ssum + g_ref[...]) + b_ref[...]
        t_ref[...] = t
        z_ref[...] = _dot(t, w_ref[...])

    return pl.pallas_call(
        body,
        out_shape=(jax.ShapeDtypeStruct((N, H), jnp.float32),
                   jax.ShapeDtypeStruct((N, H), jnp.float32)),
    )(s_il, g2, dinv, b2_2, Wb)


def _pad_idx(idx, pad):
    return jnp.concatenate([idx, pad]).reshape(NW, CHUNKS, K)


def kernel(x_input, edge_index_input, pos_edge_index_input,
           W_init, b_init, W1, b1, W2, b2, W_bil, b_bil):
    pad_iota = jnp.arange(EP - E, dtype=jnp.int32)
    pad_src = pad_iota % N
    pad_dst = N + pad_iota % (ACC_N - N)
    rows3 = _pad_idx(pos_edge_index_input[0], pad_src)
    cols3 = _pad_idx(pos_edge_index_input[1], pad_dst)
    e0_3 = _pad_idx(edge_index_input[0], pad_src)
    e1_3 = _pad_idx(edge_index_input[1], pad_src)
    zero16 = jnp.zeros((ACC_N, 16), jnp.float32)
    zero64 = jnp.zeros((ACC_N, H), jnp.float32)

    hist = _sc_hist(cols3, zero16)
    h1 = _tc_mm1(x_input, W_init, b_init.reshape(1, F), W1)
    g1, dinv = _tc_scale(h1, hist)

    s1 = _sc_agg(g1, rows3, cols3, zero64)
    g2 = _tc_mid(s1, g1, dinv, b1.reshape(1, H), W2)

    s2 = _sc_agg(g2, rows3, cols3, zero64)
    out2, z = _tc_fin(s2, g2, dinv, b2.reshape(1, H), W_bil[0])

    yp = _sc_dot(z, out2, e0_3, e1_3)
    return yp.reshape(-1)[:E] + b_bil[0]
